# baseline (speedup 1.0000x reference)
import jax
import jax.numpy as jnp
from jax import lax
from jax.experimental import pallas as pl
from jax.experimental.pallas import tpu as pltpu
from jax.experimental.pallas import tpu_sc as plsc

HIDDEN = 768
SEQ = 512
NV = HIDDEN // 16
NC, NS = 2, 16
NW = NC * NS
EPS = 1e-12


def _lane_sum(x):
    for s in (1, 2, 4, 8):
        idx = lax.iota(jnp.int32, 16) ^ s
        perm = lax.gather(
            x, idx[:, None],
            lax.GatherDimensionNumbers(offset_dims=(),
                                       collapsed_slice_dims=(0,),
                                       start_index_map=(0,)),
            slice_sizes=(1,),
            mode=lax.GatherScatterMode.PROMISE_IN_BOUNDS)
        x = x + perm
    return x


def _sc_embed_ln(ids_hbm, wtab_hbm, ptab_hbm, gamma_hbm,
                 beta_hbm, tt_hbm, out_hbm, idsbuf, ttbuf, ptidxbuf,
                 wbuf, pbuf, gbuf, bbuf, sem1, sem2):
    ntok = out_hbm.shape[0]
    tok_per_w = ntok // NW
    ch = 32
    nchunk = tok_per_w // ch
    wid = lax.axis_index("s") * NC + lax.axis_index("c")
    base0 = wid * tok_per_w

    pltpu.sync_copy(gamma_hbm, gbuf)
    pltpu.sync_copy(beta_hbm, bbuf)

    def chunk_body(c, carry):
        base = base0 + c * ch
        posbase = lax.rem(base, SEQ)
        pltpu.sync_copy(ids_hbm.at[pl.ds(base, ch)], idsbuf)
        pltpu.sync_copy(tt_hbm.at[pl.ds(base, ch)], ttbuf)
        for k in range(ch // 16):
            ttv = ttbuf[pl.ds(k * 16, 16)]
            posv = posbase + k * 16 + lax.iota(jnp.int32, 16)
            ptidxbuf[pl.ds(k * 16, 16)] = posv + SEQ * ttv
        cp1 = pltpu.async_copy(wtab_hbm.at[idsbuf], wbuf, sem1)
        cp2 = pltpu.async_copy(ptab_hbm.at[ptidxbuf], pbuf, sem2)
        cp1.wait()
        cp2.wait()

        def tok_body(t, _):
            acc = jnp.zeros((16,), jnp.float32)
            accsq = jnp.zeros((16,), jnp.float32)
            xs = []
            for j in range(NV):
                x = wbuf[t, pl.ds(j * 16, 16)] + pbuf[t, pl.ds(j * 16, 16)]
                acc = acc + x
                accsq = accsq + x * x
                xs.append(x)
            us = _lane_sum(acc) * (1.0 / HIDDEN)
            v = _lane_sum(accsq) * (1.0 / HIDDEN) - us * us + EPS
            yi = jnp.int32(0x5F3759DF) - lax.shift_right_logical(
                lax.bitcast_convert_type(v, jnp.int32), 1)
            y = lax.bitcast_convert_type(yi, jnp.float32)
            for _ in range(3):
                y = y * (1.5 - 0.5 * v * y * y)
            for j in range(NV):
                g = gbuf[pl.ds(j * 16, 16)]
                b = bbuf[pl.ds(j * 16, 16)]
                pbuf[t, pl.ds(j * 16, 16)] = (xs[j] - us) * y * g + b
            return _

        lax.fori_loop(0, ch, tok_body, 0)
        pltpu.sync_copy(pbuf, out_hbm.at[pl.ds(base, ch)])
        return carry

    lax.fori_loop(0, nchunk, chunk_body, 0)


def kernel(input_ids, token_type_ids, word_embeddings, position_embeddings,
           token_type_embeddings, gamma, beta):
    batch, seq = input_ids.shape
    ntok = batch * seq
    ids_flat = input_ids.reshape(ntok).astype(jnp.int32)
    tt_flat = token_type_ids.reshape(ntok).astype(jnp.int32)
    ptab = jnp.concatenate(
        [position_embeddings + token_type_embeddings[0][None, :],
         position_embeddings + token_type_embeddings[1][None, :]], axis=0)

    ch = 32
    mesh = plsc.VectorSubcoreMesh(core_axis_name="c", subcore_axis_name="s",
                                  num_cores=NC, num_subcores=NS)
    run = pl.kernel(
        _sc_embed_ln,
        out_type=jax.ShapeDtypeStruct((ntok, HIDDEN), jnp.float32),
        mesh=mesh,
        scratch_types=[
            pltpu.VMEM((ch,), jnp.int32),
            pltpu.VMEM((ch,), jnp.int32),
            pltpu.VMEM((ch,), jnp.int32),
            pltpu.VMEM((ch, HIDDEN), jnp.float32),
            pltpu.VMEM((ch, HIDDEN), jnp.float32),
            pltpu.VMEM((HIDDEN,), jnp.float32),
            pltpu.VMEM((HIDDEN,), jnp.float32),
            pltpu.SemaphoreType.DMA,
            pltpu.SemaphoreType.DMA,
        ],
    )
    out = run(ids_flat, word_embeddings, ptab, gamma, beta, tt_flat)
    return out.reshape(batch, seq, HIDDEN)
